# baseline (speedup 1.0000x reference)
import functools

import jax
import jax.numpy as jnp
from jax import lax
from jax.experimental import pallas as pl
from jax.experimental.pallas import tpu as pltpu

_OUT = 12
_S = _OUT ** 3
_G = 14
_CH = 24
_NB = 8
_TR = 392
_QUARTERS = tuple((r, r + 8) for r in range(0, _TR, 8))


def _pool_kernel(params_ref, x_ref, y_ref, z_ref, f_ref, outp_ref, outr_ref,
                 *acc_refs, n_rois, n_tiles):
    i = pl.program_id(0)
    cx = params_ref[i, 0]
    cy = params_ref[i, 1]
    cz = params_ref[i, 2]
    co = params_ref[i, 3]
    sn = params_ref[i, 4]
    hx = params_ref[i, 5]
    hy = params_ref[i, 6]
    hz = params_ref[i, 7]
    vsx = params_ref[i, 8]
    vsy = params_ref[i, 9]
    vsz = params_ref[i, 10]

    pos = (lax.broadcasted_iota(jnp.int32, (_TR, 128), 0) * 128
           + lax.broadcasted_iota(jnp.int32, (_TR, 128), 1))
    lane32 = lax.broadcasted_iota(jnp.int32, (_CH, 128), 1)
    is_sum = lax.broadcasted_iota(jnp.int32, (_CH, 1), 0) < 5
    neutral = jnp.where(is_sum, 0.0, -jnp.inf).astype(jnp.float32)

    init = jnp.where(
        lax.broadcasted_iota(jnp.int32, (_G, _CH, 128), 1) < 5,
        0.0, -jnp.inf).astype(jnp.float32)
    for a in acc_refs:
        a[...] = init

    for t in range(n_tiles):
        xs = x_ref[t]
        ys = y_ref[t]
        zs = z_ref[t]
        sx = xs - cx
        sy = ys - cy
        lz = zs - cz
        lx = sx * co + sy * sn
        ly = -sx * sn + sy * co
        inb = ((jnp.abs(lx) < hx) & (jnp.abs(ly) < hy)
               & (jnp.abs(lz) < hz))
        vx = jnp.clip(jnp.floor((lx + hx) / vsx).astype(jnp.int32), 0, _OUT - 1)
        vy = jnp.clip(jnp.floor((ly + hy) / vsy).astype(jnp.int32), 0, _OUT - 1)
        vz = jnp.clip(jnp.floor((lz + hz) / vsz).astype(jnp.int32), 0, _OUT - 1)
        seg = (vx * _OUT + vy) * _OUT + vz
        key = jnp.where(inb, seg * 65536 + pos, -1)
        inbf = inb.astype(jnp.float32)
        nq = [jnp.sum(inbf[r0:r1, :]).astype(jnp.int32)
              for r0, r1 in _QUARTERS]
        nmax = functools.reduce(jnp.maximum, nq)

        def body(_, carry, t=t):
            new_carry = []
            for h, kh in enumerate(carry):
                q = jnp.max(kh)
                valid = q >= 0
                sseg = jnp.maximum(q >> 16, 0)
                flat = jnp.where(valid, q & 65535, 0)
                q_lane = flat & 127
                p_row = t * _TR + (flat >> 7)
                fblk = f_ref[p_row]
                contrib = jnp.sum(jnp.where(lane32 == q_lane, fblk, 0.0),
                                  axis=1, keepdims=True)
                contrib = jnp.where(valid, contrib, neutral)
                g = sseg >> 7
                l = sseg & 127
                a_ref = acc_refs[h % _NB]
                ablk = a_ref[g]
                upd = jnp.where(is_sum, ablk + contrib,
                                jnp.maximum(ablk, contrib))
                a_ref[g] = jnp.where(lane32 == l, upd, ablk)
                new_carry.append(jnp.where(kh == q, -1, kh))
            return tuple(new_carry)

        lax.fori_loop(0, nmax, body,
                      tuple(key[r0:r1, :] for r0, r1 in _QUARTERS))

    for g in range(_G):
        blks = [a[g] for a in acc_refs]
        ssum = blks[0][0:5, :]
        smax = blks[0][5:21, :]
        for b in blks[1:]:
            ssum = ssum + b[0:5, :]
            smax = jnp.maximum(smax, b[5:21, :])
        cnt = ssum[4:5, :]
        outp_ref[0, :, 128 * g:128 * (g + 1)] = (
            ssum[0:4, :] / jnp.maximum(cnt, 1.0))
        outr_ref[0, :, 128 * g:128 * (g + 1)] = jnp.where(
            cnt > 0, smax, 0.0)


def kernel(rois, voxel_centers, part_features, rpn_features):
    B, N = rois.shape[0], rois.shape[1]
    M = voxel_centers.shape[1]
    NT = -(-M // (128 * _TR))
    Mpad = NT * _TR * 128
    R = NT * _TR
    BN = B * N
    f32 = jnp.float32

    pad = Mpad - M
    pcoord = jnp.pad(voxel_centers, ((0, 0), (0, pad), (0, 0)),
                     constant_values=1e9)
    x3 = pcoord[..., 0].reshape(B * NT, _TR, 128)
    y3 = pcoord[..., 1].reshape(B * NT, _TR, 128)
    z3 = pcoord[..., 2].reshape(B * NT, _TR, 128)

    ones = jnp.ones((B, M, 1), f32)
    zpad = jnp.zeros((B, M, _CH - 21), f32)
    fall = jnp.concatenate([part_features, ones, rpn_features, zpad], axis=-1)
    fall = jnp.pad(fall, ((0, 0), (0, pad), (0, 0)))
    feats3 = fall.reshape(B, R, 128, _CH).transpose(0, 1, 3, 2).reshape(
        B * R, _CH, 128)

    centers = rois[..., 0:3]
    dims = rois[..., 3:6]
    rz = rois[..., 6:7]
    half = dims * 0.5
    vs = dims / _OUT
    params = jnp.concatenate(
        [centers, jnp.cos(rz), jnp.sin(rz), half, vs,
         jnp.zeros((B, N, 5), f32)], axis=-1).reshape(BN, 16)

    grid_spec = pltpu.PrefetchScalarGridSpec(
        num_scalar_prefetch=1,
        grid=(BN,),
        in_specs=[
            pl.BlockSpec((NT, _TR, 128), lambda i, p: (i // N, 0, 0)),
            pl.BlockSpec((NT, _TR, 128), lambda i, p: (i // N, 0, 0)),
            pl.BlockSpec((NT, _TR, 128), lambda i, p: (i // N, 0, 0)),
            pl.BlockSpec((R, _CH, 128), lambda i, p: (i // N, 0, 0)),
        ],
        out_specs=[
            pl.BlockSpec((1, 4, 128 * _G), lambda i, p: (i, 0, 0)),
            pl.BlockSpec((1, 16, 128 * _G), lambda i, p: (i, 0, 0)),
        ],
        scratch_shapes=[pltpu.VMEM((_G, _CH, 128), f32)
                        for _ in range(_NB)],
    )
    outp, outr = pl.pallas_call(
        functools.partial(_pool_kernel, n_rois=N, n_tiles=NT),
        grid_spec=grid_spec,
        out_shape=(
            jax.ShapeDtypeStruct((BN, 4, 128 * _G), f32),
            jax.ShapeDtypeStruct((BN, 16, 128 * _G), f32),
        ),
        compiler_params=pltpu.CompilerParams(
            dimension_semantics=("parallel",)),
    )(params, x3, y3, z3, feats3)

    pooled_part = outp[:, :, :_S].transpose(0, 2, 1).reshape(
        BN, _OUT, _OUT, _OUT, 4)
    pooled_rpn = outr[:, :, :_S].transpose(0, 2, 1).reshape(
        BN, _OUT, _OUT, _OUT, 16)
    return pooled_part, pooled_rpn


# speedup vs baseline: 1.0222x; 1.0222x over previous
import functools

import jax
import jax.numpy as jnp
from jax import lax
from jax.experimental import pallas as pl
from jax.experimental.pallas import tpu as pltpu

_OUT = 12
_S = _OUT ** 3
_G = 14
_CH = 24
_NB = 4
_TR = 392
_QUARTERS = tuple((r, r + 8) for r in range(0, _TR, 8))


def _pool_kernel(params_ref, x_ref, y_ref, z_ref, f_ref, outp_ref, outr_ref,
                 *acc_refs, n_rois, n_tiles):
    i = pl.program_id(0)
    cx = params_ref[i, 0]
    cy = params_ref[i, 1]
    cz = params_ref[i, 2]
    co = params_ref[i, 3]
    sn = params_ref[i, 4]
    hx = params_ref[i, 5]
    hy = params_ref[i, 6]
    hz = params_ref[i, 7]
    ivx = params_ref[i, 8]
    ivy = params_ref[i, 9]
    ivz = params_ref[i, 10]

    pos = (lax.broadcasted_iota(jnp.int32, (_TR, 128), 0) * 128
           + lax.broadcasted_iota(jnp.int32, (_TR, 128), 1))
    lane32 = lax.broadcasted_iota(jnp.int32, (_CH, 128), 1)
    is_sum = lax.broadcasted_iota(jnp.int32, (_CH, 1), 0) < 5
    neutral = jnp.where(is_sum, 0.0, -jnp.inf).astype(jnp.float32)

    init = jnp.where(
        lax.broadcasted_iota(jnp.int32, (_G, _CH, 128), 1) < 5,
        0.0, -jnp.inf).astype(jnp.float32)
    for a in acc_refs:
        a[...] = init

    for t in range(n_tiles):
        xs = x_ref[t]
        ys = y_ref[t]
        zs = z_ref[t]
        sx = xs - cx
        sy = ys - cy
        lz = zs - cz
        lx = sx * co + sy * sn
        ly = -sx * sn + sy * co
        inb = ((jnp.abs(lx) < hx) & (jnp.abs(ly) < hy)
               & (jnp.abs(lz) < hz))
        vx = jnp.clip(jnp.floor((lx + hx) * ivx).astype(jnp.int32), 0, _OUT - 1)
        vy = jnp.clip(jnp.floor((ly + hy) * ivy).astype(jnp.int32), 0, _OUT - 1)
        vz = jnp.clip(jnp.floor((lz + hz) * ivz).astype(jnp.int32), 0, _OUT - 1)
        seg = (vx * _OUT + vy) * _OUT + vz
        key = jnp.where(inb, seg * 65536 + pos, -1)
        inbf = inb.astype(jnp.float32)
        nq = [jnp.sum(inbf[r0:r1, :]).astype(jnp.int32)
              for r0, r1 in _QUARTERS]
        nmax = functools.reduce(jnp.maximum, nq)

        def body(_, carry, t=t):
            new_carry = []
            for h, kh in enumerate(carry):
                q = jnp.max(kh)
                valid = q >= 0
                sseg = jnp.maximum(q >> 16, 0)
                flat = jnp.where(valid, q & 65535, 0)
                q_lane = flat & 127
                p_row = t * _TR + (flat >> 7)
                fblk = f_ref[p_row]
                contrib = jnp.sum(jnp.where(lane32 == q_lane, fblk, 0.0),
                                  axis=1, keepdims=True)
                contrib = jnp.where(valid, contrib, neutral)
                g = sseg >> 7
                l = sseg & 127
                a_ref = acc_refs[h % _NB]
                ablk = a_ref[g]
                upd = jnp.where(is_sum, ablk + contrib,
                                jnp.maximum(ablk, contrib))
                a_ref[g] = jnp.where(lane32 == l, upd, ablk)
                new_carry.append(jnp.where(kh == q, -1, kh))
            return tuple(new_carry)

        lax.fori_loop(0, nmax, body,
                      tuple(key[r0:r1, :] for r0, r1 in _QUARTERS))

    for g in range(_G):
        blks = [a[g] for a in acc_refs]
        ssum = blks[0][0:5, :]
        smax = blks[0][5:21, :]
        for b in blks[1:]:
            ssum = ssum + b[0:5, :]
            smax = jnp.maximum(smax, b[5:21, :])
        cnt = ssum[4:5, :]
        outp_ref[0, :, 128 * g:128 * (g + 1)] = (
            ssum[0:4, :] / jnp.maximum(cnt, 1.0))
        outr_ref[0, :, 128 * g:128 * (g + 1)] = jnp.where(
            cnt > 0, smax, 0.0)


def kernel(rois, voxel_centers, part_features, rpn_features):
    B, N = rois.shape[0], rois.shape[1]
    M = voxel_centers.shape[1]
    NT = -(-M // (128 * _TR))
    Mpad = NT * _TR * 128
    R = NT * _TR
    BN = B * N
    f32 = jnp.float32

    pad = Mpad - M
    pcoord = jnp.pad(voxel_centers, ((0, 0), (0, pad), (0, 0)),
                     constant_values=1e9)
    x3 = pcoord[..., 0].reshape(B * NT, _TR, 128)
    y3 = pcoord[..., 1].reshape(B * NT, _TR, 128)
    z3 = pcoord[..., 2].reshape(B * NT, _TR, 128)

    ones = jnp.ones((B, M, 1), f32)
    zpad = jnp.zeros((B, M, _CH - 21), f32)
    fall = jnp.concatenate([part_features, ones, rpn_features, zpad], axis=-1)
    fall = jnp.pad(fall, ((0, 0), (0, pad), (0, 0)))
    feats3 = fall.reshape(B, R, 128, _CH).transpose(0, 1, 3, 2).reshape(
        B * R, _CH, 128)

    centers = rois[..., 0:3]
    dims = rois[..., 3:6]
    rz = rois[..., 6:7]
    half = dims * 0.5
    ivs = _OUT / dims
    params = jnp.concatenate(
        [centers, jnp.cos(rz), jnp.sin(rz), half, ivs,
         jnp.zeros((B, N, 5), f32)], axis=-1).reshape(BN, 16)

    grid_spec = pltpu.PrefetchScalarGridSpec(
        num_scalar_prefetch=1,
        grid=(BN,),
        in_specs=[
            pl.BlockSpec((NT, _TR, 128), lambda i, p: (i // N, 0, 0)),
            pl.BlockSpec((NT, _TR, 128), lambda i, p: (i // N, 0, 0)),
            pl.BlockSpec((NT, _TR, 128), lambda i, p: (i // N, 0, 0)),
            pl.BlockSpec((R, _CH, 128), lambda i, p: (i // N, 0, 0)),
        ],
        out_specs=[
            pl.BlockSpec((1, 4, 128 * _G), lambda i, p: (i, 0, 0)),
            pl.BlockSpec((1, 16, 128 * _G), lambda i, p: (i, 0, 0)),
        ],
        scratch_shapes=[pltpu.VMEM((_G, _CH, 128), f32)
                        for _ in range(_NB)],
    )
    outp, outr = pl.pallas_call(
        functools.partial(_pool_kernel, n_rois=N, n_tiles=NT),
        grid_spec=grid_spec,
        out_shape=(
            jax.ShapeDtypeStruct((BN, 4, 128 * _G), f32),
            jax.ShapeDtypeStruct((BN, 16, 128 * _G), f32),
        ),
        compiler_params=pltpu.CompilerParams(
            dimension_semantics=("parallel",)),
    )(params, x3, y3, z3, feats3)

    pooled_part = outp[:, :, :_S].transpose(0, 2, 1).reshape(
        BN, _OUT, _OUT, _OUT, 4)
    pooled_rpn = outr[:, :, :_S].transpose(0, 2, 1).reshape(
        BN, _OUT, _OUT, _OUT, 16)
    return pooled_part, pooled_rpn


# speedup vs baseline: 1.0249x; 1.0027x over previous
import functools

import jax
import jax.numpy as jnp
from jax import lax
from jax.experimental import pallas as pl
from jax.experimental.pallas import tpu as pltpu

_OUT = 12
_S = _OUT ** 3
_G = 14
_CH = 24
_NB = 4
_TR = 392
_QUARTERS = tuple((r, r + 8) for r in range(0, _TR, 8))


def _pool_kernel(params_ref, x_ref, y_ref, z_ref, f_ref, outp_ref, outr_ref,
                 *acc_refs, n_rois, n_tiles):
    i = pl.program_id(0)
    cx = params_ref[i, 0]
    cy = params_ref[i, 1]
    cz = params_ref[i, 2]
    co = params_ref[i, 3]
    sn = params_ref[i, 4]
    hx = params_ref[i, 5]
    hy = params_ref[i, 6]
    hz = params_ref[i, 7]
    vsx = params_ref[i, 8]
    vsy = params_ref[i, 9]
    vsz = params_ref[i, 10]

    pos = (lax.broadcasted_iota(jnp.int32, (_TR, 128), 0) * 128
           + lax.broadcasted_iota(jnp.int32, (_TR, 128), 1))
    lane32 = lax.broadcasted_iota(jnp.int32, (_CH, 128), 1)
    is_sum = lax.broadcasted_iota(jnp.int32, (_CH, 1), 0) < 5
    neutral = jnp.where(is_sum, 0.0, -jnp.inf).astype(jnp.float32)

    init = jnp.where(
        lax.broadcasted_iota(jnp.int32, (_G, _CH, 128), 1) < 5,
        0.0, -jnp.inf).astype(jnp.float32)
    for a in acc_refs:
        a[...] = init

    for t in range(n_tiles):
        xs = x_ref[t]
        ys = y_ref[t]
        zs = z_ref[t]
        sx = xs - cx
        sy = ys - cy
        lz = zs - cz
        lx = sx * co + sy * sn
        ly = -sx * sn + sy * co
        inb = ((jnp.abs(lx) < hx) & (jnp.abs(ly) < hy)
               & (jnp.abs(lz) < hz))
        vx = jnp.clip(jnp.floor((lx + hx) / vsx).astype(jnp.int32), 0, _OUT - 1)
        vy = jnp.clip(jnp.floor((ly + hy) / vsy).astype(jnp.int32), 0, _OUT - 1)
        vz = jnp.clip(jnp.floor((lz + hz) / vsz).astype(jnp.int32), 0, _OUT - 1)
        seg = (vx * _OUT + vy) * _OUT + vz
        key = jnp.where(inb, seg * 65536 + pos, -1)
        inbf = inb.astype(jnp.float32)
        nq = [jnp.sum(inbf[r0:r1, :]).astype(jnp.int32)
              for r0, r1 in _QUARTERS]
        nmax = functools.reduce(jnp.maximum, nq)

        def body(_, carry, t=t):
            new_carry = []
            for h, kh in enumerate(carry):
                q = jnp.max(kh)
                valid = q >= 0
                sseg = jnp.maximum(q >> 16, 0)
                flat = jnp.where(valid, q & 65535, 0)
                q_lane = flat & 127
                p_row = t * _TR + (flat >> 7)
                fblk = f_ref[p_row]
                contrib = jnp.sum(jnp.where(lane32 == q_lane, fblk, 0.0),
                                  axis=1, keepdims=True)
                contrib = jnp.where(valid, contrib, neutral)
                g = sseg >> 7
                l = sseg & 127
                a_ref = acc_refs[h % _NB]
                ablk = a_ref[g]
                upd = jnp.where(is_sum, ablk + contrib,
                                jnp.maximum(ablk, contrib))
                a_ref[g] = jnp.where(lane32 == l, upd, ablk)
                new_carry.append(jnp.where(kh == q, -1, kh))
            return tuple(new_carry)

        lax.fori_loop(0, nmax, body,
                      tuple(key[r0:r1, :] for r0, r1 in _QUARTERS))

    for g in range(_G):
        blks = [a[g] for a in acc_refs]
        ssum = blks[0][0:5, :]
        smax = blks[0][5:21, :]
        for b in blks[1:]:
            ssum = ssum + b[0:5, :]
            smax = jnp.maximum(smax, b[5:21, :])
        cnt = ssum[4:5, :]
        outp_ref[0, :, 128 * g:128 * (g + 1)] = (
            ssum[0:4, :] / jnp.maximum(cnt, 1.0))
        outr_ref[0, :, 128 * g:128 * (g + 1)] = jnp.where(
            cnt > 0, smax, 0.0)


def kernel(rois, voxel_centers, part_features, rpn_features):
    B, N = rois.shape[0], rois.shape[1]
    M = voxel_centers.shape[1]
    NT = -(-M // (128 * _TR))
    Mpad = NT * _TR * 128
    R = NT * _TR
    BN = B * N
    f32 = jnp.float32

    pad = Mpad - M
    pcoord = jnp.pad(voxel_centers, ((0, 0), (0, pad), (0, 0)),
                     constant_values=1e9)
    x3 = pcoord[..., 0].reshape(B * NT, _TR, 128)
    y3 = pcoord[..., 1].reshape(B * NT, _TR, 128)
    z3 = pcoord[..., 2].reshape(B * NT, _TR, 128)

    ones = jnp.ones((B, M, 1), f32)
    zpad = jnp.zeros((B, M, _CH - 21), f32)
    fall = jnp.concatenate([part_features, ones, rpn_features, zpad], axis=-1)
    fall = jnp.pad(fall, ((0, 0), (0, pad), (0, 0)))
    feats3 = fall.reshape(B, R, 128, _CH).transpose(0, 1, 3, 2).reshape(
        B * R, _CH, 128)

    centers = rois[..., 0:3]
    dims = rois[..., 3:6]
    rz = rois[..., 6:7]
    half = dims * 0.5
    vs = dims / _OUT
    params = jnp.concatenate(
        [centers, jnp.cos(rz), jnp.sin(rz), half, vs,
         jnp.zeros((B, N, 5), f32)], axis=-1).reshape(BN, 16)

    grid_spec = pltpu.PrefetchScalarGridSpec(
        num_scalar_prefetch=1,
        grid=(BN,),
        in_specs=[
            pl.BlockSpec((NT, _TR, 128), lambda i, p: (i // N, 0, 0)),
            pl.BlockSpec((NT, _TR, 128), lambda i, p: (i // N, 0, 0)),
            pl.BlockSpec((NT, _TR, 128), lambda i, p: (i // N, 0, 0)),
            pl.BlockSpec((R, _CH, 128), lambda i, p: (i // N, 0, 0)),
        ],
        out_specs=[
            pl.BlockSpec((1, 4, 128 * _G), lambda i, p: (i, 0, 0)),
            pl.BlockSpec((1, 16, 128 * _G), lambda i, p: (i, 0, 0)),
        ],
        scratch_shapes=[pltpu.VMEM((_G, _CH, 128), f32)
                        for _ in range(_NB)],
    )
    outp, outr = pl.pallas_call(
        functools.partial(_pool_kernel, n_rois=N, n_tiles=NT),
        grid_spec=grid_spec,
        out_shape=(
            jax.ShapeDtypeStruct((BN, 4, 128 * _G), f32),
            jax.ShapeDtypeStruct((BN, 16, 128 * _G), f32),
        ),
        compiler_params=pltpu.CompilerParams(
            dimension_semantics=("parallel",)),
    )(params, x3, y3, z3, feats3)

    pooled_part = outp[:, :, :_S].transpose(0, 2, 1).reshape(
        BN, _OUT, _OUT, _OUT, 4)
    pooled_rpn = outr[:, :, :_S].transpose(0, 2, 1).reshape(
        BN, _OUT, _OUT, _OUT, 16)
    return pooled_part, pooled_rpn
